# Initial kernel scaffold; baseline (speedup 1.0000x reference)
import jax
import jax.numpy as jnp
from jax.experimental import pallas as pl


def kernel(x, neighbors, Wq, bq, Wk, bk, Wv, bv, Wo, bo):
    raise NotImplementedError("write your pallas kernel here")


# speedup vs baseline: 1.9294x; 1.9294x over previous
import functools

import jax
import jax.numpy as jnp
from jax import lax
from jax.experimental import pallas as pl
from jax.experimental.pallas import tpu as pltpu
from jax.experimental.pallas import tpu_sc as plsc

DIM = 128
H = 4
HD = DIM // H
KN = 32
L = 16
N_PAD = 10240
NW = 32
NPW = N_PAD // NW
C = 4
CH = NPW // C
SCALE = 1.0 / (HD ** 0.5)


def _proj_body(x_ref, wq_ref, bq_ref, wk_ref, bk_ref, wv_ref, bv_ref,
               q_ref, kv_ref):
    xb = x_ref[...]
    dn = (((1,), (1,)), ((), ()))
    q_ref[...] = lax.dot_general(xb, wq_ref[...], dn,
                                 preferred_element_type=jnp.float32) + bq_ref[...]
    kv_ref[:, :DIM] = lax.dot_general(xb, wk_ref[...], dn,
                                      preferred_element_type=jnp.float32) + bk_ref[...]
    kv_ref[:, DIM:] = lax.dot_general(xb, wv_ref[...], dn,
                                      preferred_element_type=jnp.float32) + bv_ref[...]


def _project(x_pad, Wq, bq, Wk, bk, Wv, bv):
    blk = 1024
    grid = (N_PAD // blk,)
    full = pl.BlockSpec((DIM, DIM), lambda i: (0, 0))
    bias = pl.BlockSpec((1, DIM), lambda i: (0, 0))
    return pl.pallas_call(
        _proj_body,
        grid=grid,
        in_specs=[
            pl.BlockSpec((blk, DIM), lambda i: (i, 0)),
            full, bias, full, bias, full, bias,
        ],
        out_specs=[
            pl.BlockSpec((blk, DIM), lambda i: (i, 0)),
            pl.BlockSpec((blk, 2 * DIM), lambda i: (i, 0)),
        ],
        out_shape=[
            jax.ShapeDtypeStruct((N_PAD, DIM), jnp.float32),
            jax.ShapeDtypeStruct((N_PAD, 2 * DIM), jnp.float32),
        ],
    )(x_pad, Wq, bq.reshape(1, DIM), Wk, bk.reshape(1, DIM),
      Wv, bv.reshape(1, DIM))


def _outproj_body(a_ref, wo_ref, bo_ref, o_ref):
    dn = (((1,), (1,)), ((), ()))
    o_ref[...] = lax.dot_general(a_ref[...], wo_ref[...], dn,
                                 preferred_element_type=jnp.float32) + bo_ref[...]


def _outproj(att, Wo, bo):
    blk = 1024
    return pl.pallas_call(
        _outproj_body,
        grid=(N_PAD // blk,),
        in_specs=[
            pl.BlockSpec((blk, DIM), lambda i: (i, 0)),
            pl.BlockSpec((DIM, DIM), lambda i: (0, 0)),
            pl.BlockSpec((1, DIM), lambda i: (0, 0)),
        ],
        out_specs=pl.BlockSpec((blk, DIM), lambda i: (i, 0)),
        out_shape=jax.ShapeDtypeStruct((N_PAD, DIM), jnp.float32),
    )(att, Wo, bo.reshape(1, DIM))


def _node_attention(qb, kvb, pb, ob, n):
    iota = lax.iota(jnp.int32, L)
    zeros = jnp.zeros((L,), jnp.float32)
    qv = [[qb[pl.ds(n * DIM + h * HD + j * L, L)] for j in range(2)]
          for h in range(H)]

    def sbody(kk, carry):
        s = list(carry)
        row = n * KN + kk
        for h in range(H):
            klo = kvb[row, pl.ds(h * HD, L)]
            khi = kvb[row, pl.ds(h * HD + L, L)]
            t = qv[h][0] * klo + qv[h][1] * khi
            sc = jnp.sum(t) * SCALE
            bc = jnp.full((L,), sc, jnp.float32)
            s[2 * h] = jnp.where(iota == kk, bc, s[2 * h])
            s[2 * h + 1] = jnp.where(iota == (kk - L), bc, s[2 * h + 1])
        return tuple(s)

    svecs = lax.fori_loop(0, KN, sbody, (zeros,) * (2 * H))

    for h in range(H):
        slo, shi = svecs[2 * h], svecs[2 * h + 1]
        m = jnp.maximum(jnp.max(slo), jnp.max(shi))
        elo = jnp.exp(slo - m)
        ehi = jnp.exp(shi - m)
        zv = jnp.full((L,), jnp.sum(elo) + jnp.sum(ehi), jnp.float32)
        inv = jnp.full((L,), 1.0, jnp.float32) / zv
        pb[pl.ds(h * KN, L)] = elo * inv
        pb[pl.ds(h * KN + L, L)] = ehi * inv

    def abody(kk, carry):
        a = list(carry)
        row = n * KN + kk
        for h in range(H):
            pvec = plsc.load_gather(pb, [jnp.full((L,), h * KN, jnp.int32) + kk])
            vlo = kvb[row, pl.ds(DIM + h * HD, L)]
            vhi = kvb[row, pl.ds(DIM + h * HD + L, L)]
            a[2 * h] = a[2 * h] + pvec * vlo
            a[2 * h + 1] = a[2 * h + 1] + pvec * vhi
        return tuple(a)

    avecs = lax.fori_loop(0, KN, abody, (zeros,) * (2 * H))
    for h in range(H):
        for j in range(2):
            ob[pl.ds(n * DIM + h * HD + j * L, L)] = avecs[2 * h + j]


def _sc_attention(q, kv, nbr):
    mesh = plsc.VectorSubcoreMesh(core_axis_name="c", subcore_axis_name="s")

    @functools.partial(
        pl.kernel,
        out_type=jax.ShapeDtypeStruct((N_PAD * DIM,), jnp.float32),
        mesh=mesh,
        compiler_params=pltpu.CompilerParams(needs_layout_passes=False),
        scratch_types=[
            pltpu.VMEM((C * KN,), jnp.int32),
            pltpu.VMEM((C * KN,), jnp.int32),
            pltpu.VMEM((C * KN, 2 * DIM), jnp.float32),
            pltpu.VMEM((C * KN, 2 * DIM), jnp.float32),
            pltpu.VMEM((C * DIM,), jnp.float32),
            pltpu.VMEM((C * DIM,), jnp.float32),
            pltpu.VMEM((C * DIM,), jnp.float32),
            pltpu.VMEM((H * KN,), jnp.float32),
            pltpu.SemaphoreType.DMA,
            pltpu.SemaphoreType.DMA,
        ],
    )
    def run(q_hbm, kv_hbm, nbr_hbm, out_hbm,
            idx0, idx1, kv0, kv1, q0, q1, ob, pb, sem0, sem1):
        wid = lax.axis_index("s") * 2 + lax.axis_index("c")
        base = wid * NPW

        def start(g, idxb, kvb, qb, sem):
            nb = base + g * C
            pltpu.sync_copy(nbr_hbm.at[pl.ds(nb * KN, C * KN)], idxb)
            pltpu.async_copy(kv_hbm.at[idxb], kvb, sem)
            pltpu.sync_copy(q_hbm.at[pl.ds(nb * DIM, C * DIM)], qb)

        def compute(g, idxb, kvb, qb, sem):
            pltpu.make_async_copy(kv_hbm.at[idxb], kvb, sem).wait()
            for n in range(C):
                _node_attention(qb, kvb, pb, ob, n)
            nb = base + g * C
            pltpu.sync_copy(ob, out_hbm.at[pl.ds(nb * DIM, C * DIM)])

        start(0, idx0, kv0, q0, sem0)

        def body(gg, _):
            g0 = gg * 2
            start(g0 + 1, idx1, kv1, q1, sem1)
            compute(g0, idx0, kv0, q0, sem0)

            @pl.when(g0 + 2 < CH)
            def _():
                start(g0 + 2, idx0, kv0, q0, sem0)

            compute(g0 + 1, idx1, kv1, q1, sem1)
            return 0

        lax.fori_loop(0, CH // 2, body, 0)

    return run(q, kv, nbr)


def kernel(x, neighbors, Wq, bq, Wk, bk, Wv, bv, Wo, bo):
    B, N, D = x.shape
    x2 = x.reshape(N, D)
    x_pad = jnp.pad(x2, ((0, N_PAD - N), (0, 0)))
    nbr = jnp.pad(jnp.clip(neighbors, 0, None).astype(jnp.int32),
                  ((0, N_PAD - N), (0, 0))).reshape(-1)
    q, kv = _project(x_pad, Wq, bq, Wk, bk, Wv, bv)
    att = _sc_attention(q.reshape(-1), kv, nbr)
    out = _outproj(att.reshape(N_PAD, DIM), Wo, bo)
    return out[:N].reshape(B, N, D)
